# baseline (speedup 1.0000x reference)
import functools

import jax
import jax.numpy as jnp
import numpy as np
from jax import lax
from jax.experimental import pallas as pl
from jax.experimental.pallas import tpu as pltpu
from jax.experimental.pallas import tpu_sc as plsc

V = 1000000
DIM = 64
B = 4096
S = 200
NC, NS = 2, 16
NW = NC * NS
PV = V // 2
NT = V // 128
NT_MAIN = (NT // NW) * NW
VT_PER_W = NT // NW
BBLK = B // NW

_mesh = plsc.VectorSubcoreMesh(core_axis_name="c", subcore_axis_name="s")
_params = pltpu.CompilerParams(use_tc_tiling_on_sc=True,
                               needs_layout_passes=False)

_IOTA = np.arange(16, dtype=np.int32)


def _i32(x):
    return jnp.int32(x)


@functools.partial(
    pl.kernel,
    mesh=_mesh,
    out_type=jax.ShapeDtypeStruct((PV, 128), jnp.float32),
    scratch_types=[
        pltpu.VMEM((2, DIM, 128), jnp.float32),
        pltpu.VMEM((2, DIM, 128), jnp.float32),
        pltpu.VMEM((DIM, DIM), jnp.float32),
        [pltpu.SemaphoreType.DMA] * 2,
        [pltpu.SemaphoreType.DMA] * 2,
    ],
    compiler_params=_params,
)
def _compact(wt_hbm, tail_hbm, out_hbm, tbuf, cbuf, tailv, gsems, wsems):
    wid = lax.axis_index("s") * NC + lax.axis_index("c")
    iota16 = lax.iota(jnp.int32, 16)
    vt0 = wid * VT_PER_W

    def start_read(vt, b):
        pltpu.async_copy(wt_hbm.at[:, pl.ds(vt * 128, 128)], tbuf.at[b],
                         gsems[b])

    def transpose_block(b):
        for pp in range(DIM):
            for cb in range(8):
                rows = jnp.bitwise_and(iota16 + _i32(cb * 16), _i32(63))
                cols = iota16 * 0 + _i32(2 * pp + (1 if cb >= 4 else 0))
                val = plsc.load_gather(tbuf.at[b], [rows, cols])
                cbuf[b, pp, pl.ds(cb * 16, 16)] = val

    for b in range(2):
        start_read(vt0 + b, b)

    def body(r, carry):
        for b in range(2):
            j = r * 2 + b
            vt = vt0 + j
            pltpu.make_async_copy(
                wt_hbm.at[:, pl.ds(0, 128)], tbuf.at[b], gsems[b]).wait()

            @pl.when(r > 0)
            def _():
                pltpu.make_async_copy(
                    cbuf.at[b], out_hbm.at[pl.ds(0, DIM)], wsems[b]).wait()

            transpose_block(b)
            pltpu.async_copy(cbuf.at[b], out_hbm.at[pl.ds(vt * DIM, DIM)],
                             wsems[b])

            @pl.when(vt + 2 < vt0 + VT_PER_W)
            def _():
                start_read(vt + 2, b)

        return carry

    lax.fori_loop(0, VT_PER_W // 2, body, 0)
    for b in range(2):
        pltpu.make_async_copy(
            cbuf.at[b], out_hbm.at[pl.ds(0, DIM)], wsems[b]).wait()

    @pl.when(wid < NT - NT_MAIN)
    def _():
        vt = _i32(NT_MAIN) + wid
        pltpu.async_copy(wt_hbm.at[:, pl.ds(vt * 128, 128)], tbuf.at[0],
                         gsems[0])
        pltpu.make_async_copy(
            wt_hbm.at[:, pl.ds(0, 128)], tbuf.at[0], gsems[0]).wait()
        transpose_block(0)
        pltpu.async_copy(cbuf.at[0], out_hbm.at[pl.ds(vt * DIM, DIM)],
                         wsems[0])
        pltpu.make_async_copy(
            cbuf.at[0], out_hbm.at[pl.ds(0, DIM)], wsems[0]).wait()

    @pl.when(wid == 4)
    def _():
        pltpu.sync_copy(tail_hbm, tailv)
        for pp in range(32):
            for cb in range(8):
                rows = iota16 * 0 + _i32(2 * pp + (1 if cb >= 4 else 0))
                cols = jnp.bitwise_and(iota16 + _i32(cb * 16), _i32(63))
                val = plsc.load_gather(tailv, [rows, cols])
                cbuf[0, pp, pl.ds(cb * 16, 16)] = val
        pltpu.sync_copy(cbuf.at[0, pl.ds(0, 32)],
                        out_hbm.at[pl.ds(NT * DIM, 32)])


@functools.partial(
    pl.kernel,
    mesh=_mesh,
    out_type=jax.ShapeDtypeStruct((S, DIM, B), jnp.float32),
    scratch_types=[
        pltpu.VMEM((S, BBLK), jnp.int32),
        pltpu.VMEM((2, BBLK), jnp.int32),
        pltpu.VMEM((2, BBLK, 128), jnp.float32),
        pltpu.VMEM((2, DIM, BBLK), jnp.float32),
        [pltpu.SemaphoreType.DMA] * 2,
        [pltpu.SemaphoreType.DMA] * 2,
    ],
    compiler_params=_params,
)
def _lookup(idxt_hbm, packed_hbm, out_hbm, ibuf, pidx, gbuf, obuf,
            gsems, wsems):
    wid = lax.axis_index("s") * NC + lax.axis_index("c")
    iota16 = lax.iota(jnp.int32, 16)
    pltpu.sync_copy(idxt_hbm.at[:, pl.ds(wid * BBLK, BBLK)], ibuf)

    def prep_and_gather(s, b):
        for cb in range(BBLK // 16):
            v16 = ibuf[s, pl.ds(cb * 16, 16)]
            pidx[b, pl.ds(cb * 16, 16)] = lax.shift_right_logical(v16, 1)
        pltpu.async_copy(packed_hbm.at[pidx.at[b]], gbuf.at[b], gsems[b])

    def extract(s, b):
        for cb in range(BBLK // 16):
            v16 = ibuf[s, pl.ds(cb * 16, 16)]
            half = jnp.bitwise_and(v16, 1) * _i32(DIM)
            rows = iota16 + _i32(cb * 16)
            for d in range(DIM):
                val = plsc.load_gather(gbuf.at[b], [rows, half + _i32(d)])
                obuf[b, d, pl.ds(cb * 16, 16)] = val

    for b in range(2):
        prep_and_gather(b, b)

    def body(r, carry):
        for b in range(2):
            s = r * 2 + b
            pltpu.make_async_copy(
                packed_hbm.at[pidx.at[b]], gbuf.at[b], gsems[b]).wait()

            @pl.when(r > 0)
            def _():
                pltpu.make_async_copy(
                    obuf.at[b], out_hbm.at[0, :, pl.ds(0, BBLK)],
                    wsems[b]).wait()

            extract(s, b)
            pltpu.async_copy(
                obuf.at[b], out_hbm.at[s, :, pl.ds(wid * BBLK, BBLK)],
                wsems[b])

            @pl.when(s + 2 < S)
            def _():
                prep_and_gather(s + 2, b)

        return carry

    lax.fori_loop(0, S // 2, body, 0)
    for b in range(2):
        pltpu.make_async_copy(
            obuf.at[b], out_hbm.at[0, :, pl.ds(0, BBLK)], wsems[b]).wait()


def kernel(input_ids, weight):
    packed = _compact(weight.T, weight[V - DIM:])
    out_t = _lookup(input_ids.T, packed)
    return out_t.transpose(2, 0, 1)


# speedup vs baseline: 2.7525x; 2.7525x over previous
import functools

import jax
import jax.numpy as jnp
from jax import lax
from jax.experimental import pallas as pl
from jax.experimental.pallas import tpu as pltpu
from jax.experimental.pallas import tpu_sc as plsc

V = 1000000
DIM = 64
B = 4096
S = 200
NC, NS = 2, 16
NW = NC * NS
B_PER_W = B // NW
NBUF = 4
N_ROUND = B_PER_W // NBUF

VB = 8192
NSTEP = -(-V // VB)

_mesh = plsc.VectorSubcoreMesh(core_axis_name="c", subcore_axis_name="s")


def _linearize_body(wt_ref, out_ref):
    out_ref[:, 0:DIM] = jnp.transpose(wt_ref[...], (1, 0))


_linearize = pl.pallas_call(
    _linearize_body,
    grid=(NSTEP,),
    in_specs=[pl.BlockSpec((DIM, VB), lambda i: (0, i))],
    out_specs=pl.BlockSpec((VB, 128), lambda i: (i, 0)),
    out_shape=jax.ShapeDtypeStruct((V, 128), jnp.float32),
)


@functools.partial(
    pl.kernel,
    mesh=_mesh,
    out_type=jax.ShapeDtypeStruct((B, S, DIM), jnp.float32),
    scratch_types=[
        pltpu.VMEM((B_PER_W, S), jnp.int32),
        pltpu.VMEM((NBUF, S, 128), jnp.float32),
        [pltpu.SemaphoreType.DMA] * NBUF,
        [pltpu.SemaphoreType.DMA] * NBUF,
    ],
    compiler_params=pltpu.CompilerParams(use_tc_tiling_on_sc=False),
)
def _emb_lookup(idx_hbm, table_hbm, out_hbm, idx_v, rows_v, gsems, wsems):
    wid = lax.axis_index("s") * NC + lax.axis_index("c")
    pltpu.sync_copy(idx_hbm.at[pl.ds(wid * B_PER_W, B_PER_W)], idx_v)
    b_base = wid * B_PER_W

    def start_gather(j, b):
        pltpu.async_copy(table_hbm.at[idx_v.at[j]], rows_v.at[b], gsems[b])

    for b in range(NBUF):
        start_gather(b, b)

    def body(r, carry):
        j0 = r * NBUF
        for b in range(NBUF):
            pltpu.make_async_copy(
                table_hbm.at[idx_v.at[0]], rows_v.at[b], gsems[b]).wait()
            pltpu.async_copy(
                rows_v.at[b, :, pl.ds(0, DIM)],
                out_hbm.at[b_base + j0 + b], wsems[b])
        for b in range(NBUF):
            pltpu.make_async_copy(
                rows_v.at[b, :, pl.ds(0, DIM)], out_hbm.at[0],
                wsems[b]).wait()
            jn = j0 + b + NBUF

            @pl.when(jn < B_PER_W)
            def _():
                start_gather(jn, b)

        return carry

    lax.fori_loop(0, N_ROUND, body, 0)


def kernel(input_ids, weight):
    wt = weight.T
    table = _linearize(wt)
    return _emb_lookup(input_ids, table)
